# baseline (speedup 1.0000x reference)
import functools

import jax
import jax.numpy as jnp
from jax import lax
from jax.experimental import pallas as pl
from jax.experimental.pallas import tpu as pltpu
from jax.experimental.pallas import tpu_sc as plsc

B = 2
SEQ = 4096
D = 2048
HID = 4 * D
TOPK = int(SEQ * 0.12)
KPAD = 512
NTOK = B * SEQ
R1_BLK = 256
R2_CHUNK = 512
M_HBLK = 512

_NW = 32
_BPW = (B * KPAD) // _NW


def _copy_sc(xf):
    mesh = plsc.VectorSubcoreMesh(core_axis_name="c", subcore_axis_name="s")
    rows = NTOK // _NW

    @functools.partial(
        pl.kernel,
        mesh=mesh,
        out_type=jax.ShapeDtypeStruct((NTOK, D), jnp.float32),
    )
    def c(x_hbm, o_hbm):
        wid = lax.axis_index("s") * 2 + lax.axis_index("c")
        base = wid * rows
        pltpu.sync_copy(x_hbm.at[pl.ds(base, rows)], o_hbm.at[pl.ds(base, rows)])

    return c(xf)


def _row_exclusive_prefix(m_row):
    ch = 512
    tri = (lax.broadcasted_iota(jnp.int32, (ch, ch), 0)
           < lax.broadcasted_iota(jnp.int32, (ch, ch), 1))
    tri_b = jnp.where(tri, 1.0, 0.0).astype(jnp.bfloat16)
    parts = []
    off = jnp.float32(0.0)
    for c in range(SEQ // ch):
        chunk = m_row[:, c * ch:(c + 1) * ch]
        p = jnp.dot(chunk.astype(jnp.bfloat16), tri_b,
                    preferred_element_type=jnp.float32)
        parts.append(p + off)
        off = off + jnp.sum(chunk)
    return jnp.concatenate(parts, axis=1), off


def _r2_body(lg_ref, al_ref, selg_ref, wv_ref, aux_ref):
    iota_row = lax.broadcasted_iota(jnp.int32, (1, SEQ), 1)
    iota_row_f = iota_row.astype(jnp.float32)
    qcol = lax.broadcasted_iota(jnp.int32, (KPAD, 1), 0)
    qcol_f = qcol.astype(jnp.float32)
    masks = []
    for b in range(B):
        v_row = lg_ref[b:b + 1, :]
        bits = lax.bitcast_convert_type(v_row, jnp.int32)
        skey = jnp.where(bits >= 0, bits, bits ^ jnp.int32(0x7FFFFFFF))
        def bis(_, lohi):
            lo, hi = lohi
            mid = (lo >> 1) + (hi >> 1) + (lo & hi & 1)
            cnt = jnp.sum(jnp.where(skey > mid, 1, 0))
            big = cnt >= TOPK
            return jnp.where(big, mid + 1, lo), jnp.where(big, hi, mid)
        lo, hi = lax.fori_loop(0, 32, bis,
                               (jnp.int32(-2147483648), jnp.int32(2147483647)))
        tstar = lo
        gt_row = skey > tstar
        eq_row = skey == tstar
        c_gt = jnp.sum(jnp.where(gt_row, 1, 0))
        need_eq = jnp.float32(TOPK) - c_gt.astype(jnp.float32)
        eq_pref, _ = _row_exclusive_prefix(jnp.where(eq_row, 1.0, 0.0))
        mask_row = gt_row | (eq_row & (eq_pref < need_eq))
        masks.append(mask_row)
        pos_row, _ = _row_exclusive_prefix(jnp.where(mask_row, 1.0, 0.0))
        oh = jnp.where((pos_row == qcol_f) & mask_row, 1.0, 0.0)
        sel_col = jnp.sum(oh * iota_row_f, axis=1, keepdims=True)
        vsel_col = jnp.sum(oh * v_row, axis=1, keepdims=True)
        vsel_row = jnp.transpose(vsel_col)
        q_row_f = jnp.transpose(qcol_f)
        valid_row = q_row_f < TOPK
        cmp = valid_row & ((vsel_row > vsel_col)
                           | ((vsel_row == vsel_col) & (q_row_f < qcol_f)))
        rank_col = jnp.sum(jnp.where(cmp, 1.0, 0.0), axis=1, keepdims=True)
        rank_col = jnp.where(qcol_f < TOPK, rank_col, 9999.0)
        rank_row = jnp.transpose(rank_col)
        oh2 = jnp.where(rank_row == qcol_f, 1.0, 0.0)
        wv_col = jnp.sum(oh2 * vsel_row, axis=1, keepdims=True)
        sel0 = jnp.sum(jnp.where(qcol_f == 0.0, sel_col, 0.0), axis=0,
                       keepdims=True)
        wv0 = jnp.sum(jnp.where(qcol_f == 0.0, wv_col, 0.0), axis=0,
                      keepdims=True)
        in_k = qcol_f < TOPK
        sel_col = jnp.where(in_k, sel_col, sel0)
        wv_col = jnp.where(in_k, wv_col, wv0)
        selg_ref[:, b:b + 1] = (sel_col + jnp.float32(b * SEQ)).astype(jnp.int32)
        wv_ref[:, b:b + 1] = wv_col
    al = al_ref[...]
    p = jnp.clip(1.0 / (1.0 + jnp.exp(-al)), 1e-7, 1.0 - 1e-7)
    log1m = jnp.log(1.0 - p)
    s_all = jnp.sum(jnp.sum(log1m, axis=1, keepdims=True), axis=0,
                    keepdims=True)
    union = masks[0] | masks[1]
    c0 = jnp.log(p[0:1, :]) - log1m[0:1, :]
    corr = jnp.sum(jnp.where(union, c0, 0.0), axis=1, keepdims=True)
    aux_ref[...] = -(s_all + corr) / jnp.float32(NTOK)


def _r2(lg, al):
    return pl.pallas_call(
        _r2_body,
        grid=(1,),
        in_specs=[
            pl.BlockSpec((B, SEQ), lambda i: (0, 0)),
            pl.BlockSpec((B, SEQ), lambda i: (0, 0)),
        ],
        out_specs=[
            pl.BlockSpec((KPAD, B), lambda i: (0, 0)),
            pl.BlockSpec((KPAD, B), lambda i: (0, 0)),
            pl.BlockSpec((1, 1), lambda i: (0, 0)),
        ],
        out_shape=[
            jax.ShapeDtypeStruct((KPAD, B), jnp.int32),
            jax.ShapeDtypeStruct((KPAD, B), jnp.float32),
            jax.ShapeDtypeStruct((1, 1), jnp.float32),
        ],
    )(lg, al)


def _gather(xf, idx_flat):
    mesh = plsc.VectorSubcoreMesh(core_axis_name="c", subcore_axis_name="s")

    @functools.partial(
        pl.kernel,
        mesh=mesh,
        out_type=jax.ShapeDtypeStruct((B * KPAD, D), jnp.float32),
        scratch_types=[
            pltpu.VMEM((_BPW,), jnp.int32),
            pltpu.VMEM((_BPW, D), jnp.float32),
            pltpu.SemaphoreType.DMA,
        ],
    )
    def g(x_hbm, idx_hbm, fx_hbm, idx_v, rows_v, sem):
        wid = lax.axis_index("s") * 2 + lax.axis_index("c")
        base = wid * _BPW
        pltpu.sync_copy(idx_hbm.at[pl.ds(base, _BPW)], idx_v)
        pltpu.async_copy(x_hbm.at[idx_v], rows_v, sem).wait()
        pltpu.sync_copy(rows_v, fx_hbm.at[pl.ds(base, _BPW)])

    return g(xf, idx_flat)


def _m_body(fx_ref, w1_ref, w3_ref, w2_ref, wv_ref, o_ref):
    h = pl.program_id(0)
    fxb = fx_ref[...]
    fb = fxb.astype(jnp.bfloat16)
    a = jnp.dot(fb, w1_ref[...].astype(jnp.bfloat16),
                preferred_element_type=jnp.float32)
    b3 = jnp.dot(fb, w3_ref[...].astype(jnp.bfloat16),
                 preferred_element_type=jnp.float32)
    g = a * (1.0 / (1.0 + jnp.exp(-a)))
    t = (g * b3).astype(jnp.bfloat16)
    yp = jnp.dot(t, w2_ref[...].astype(jnp.bfloat16),
                 preferred_element_type=jnp.float32)
    contrib = wv_ref[...] * yp

    @pl.when(h == 0)
    def _():
        o_ref[...] = fxb + contrib

    @pl.when(h != 0)
    def _():
        o_ref[...] = o_ref[...] + contrib


def _m(fx, wvf, W1, W2, W3):
    nh = HID // M_HBLK
    return pl.pallas_call(
        _m_body,
        grid=(nh,),
        in_specs=[
            pl.BlockSpec((B * KPAD, D), lambda h: (0, 0)),
            pl.BlockSpec((D, M_HBLK), lambda h: (0, h)),
            pl.BlockSpec((D, M_HBLK), lambda h: (0, h)),
            pl.BlockSpec((M_HBLK, D), lambda h: (h, 0)),
            pl.BlockSpec((B * KPAD, 1), lambda h: (0, 0)),
        ],
        out_specs=pl.BlockSpec((B * KPAD, D), lambda h: (0, 0)),
        out_shape=jax.ShapeDtypeStruct((B * KPAD, D), jnp.float32),
    )(fx, W1, W3, W2, wvf)


def _scatter_inplace(payload, idx_flat, out_ref):
    mesh = plsc.VectorSubcoreMesh(core_axis_name="c", subcore_axis_name="s")

    @functools.partial(
        pl.kernel,
        mesh=mesh,
        out_type=(),
        scratch_types=[
            pltpu.VMEM((_BPW,), jnp.int32),
            pltpu.VMEM((_BPW, D), jnp.float32),
            pltpu.SemaphoreType.DMA,
        ],
    )
    def s(p_hbm, idx_hbm, o_hbm, idx_v, rows_v, sem):
        wid = lax.axis_index("s") * 2 + lax.axis_index("c")
        base = wid * _BPW
        pltpu.sync_copy(idx_hbm.at[pl.ds(base, _BPW)], idx_v)
        pltpu.sync_copy(p_hbm.at[pl.ds(base, _BPW)], rows_v)
        pltpu.async_copy(rows_v, o_hbm.at[idx_v], sem).wait()

    return s(payload, idx_flat, out_ref)


def kernel(x, Wr, Wa, W1, W2, W3):
    xf = x.reshape(NTOK, D)
    lg_in = (x @ Wr)[..., 0]
    al = (xf @ Wa).reshape(B, SEQ)
    out0 = _copy_sc(xf)
    selg, wv, aux = _r2(lg_in, al)
    idx_flat = jnp.transpose(selg).reshape(B * KPAD)
    wvf = jnp.transpose(wv).reshape(B * KPAD, 1)
    fx = _gather(xf, idx_flat)
    payload = _m(fx, wvf, W1, W2, W3)
    oref = jax.new_ref(out0)
    _scatter_inplace(payload, idx_flat, oref)
    out = oref[...].reshape(B, SEQ, D)
    return out, aux[0, 0]


# speedup vs baseline: 8.1668x; 8.1668x over previous
import functools

import jax
import jax.numpy as jnp
from jax import lax
from jax.experimental import pallas as pl
from jax.experimental.pallas import tpu as pltpu
from jax.experimental.pallas import tpu_sc as plsc

B = 2
SEQ = 4096
D = 2048
HID = 4 * D
TOPK = int(SEQ * 0.12)
KPAD = 512
NTOK = B * SEQ
R1_BLK = 256
R2_CHUNK = 512
M_HBLK = 512

_NW = 32
_BPW = (B * KPAD) // _NW


def _r1_body(x_ref, wa_ref, out0_ref, al_ref):
    i = pl.program_id(0)
    per_row = SEQ // R1_BLK
    b = i // per_row
    base = pl.multiple_of((i % per_row) * R1_BLK, R1_BLK)
    xb = x_ref[...]
    out0_ref[...] = xb
    w = wa_ref[0:1, :]
    acc = xb[:, 0:128] * w[:, 0:128]
    for c in range(1, D // 128):
        acc = acc + xb[:, c * 128:(c + 1) * 128] * w[:, c * 128:(c + 1) * 128]
    z = jnp.sum(acc, axis=1, keepdims=True)
    al_ref[pl.ds(b, 1), pl.ds(base, R1_BLK)] = jnp.transpose(z)


def _r1(xf, wa_t):
    nblk = NTOK // R1_BLK
    return pl.pallas_call(
        _r1_body,
        grid=(nblk,),
        in_specs=[
            pl.BlockSpec((R1_BLK, D), lambda i: (i, 0)),
            pl.BlockSpec((1, D), lambda i: (0, 0)),
        ],
        out_specs=[
            pl.BlockSpec((R1_BLK, D), lambda i: (i, 0)),
            pl.BlockSpec((B, SEQ), lambda i: (0, 0)),
        ],
        out_shape=[
            jax.ShapeDtypeStruct((NTOK, D), jnp.float32),
            jax.ShapeDtypeStruct((B, SEQ), jnp.float32),
        ],
    )(xf, wa_t)


def _row_exclusive_prefix(m_row):
    ch = 512
    tri = (lax.broadcasted_iota(jnp.int32, (ch, ch), 0)
           < lax.broadcasted_iota(jnp.int32, (ch, ch), 1))
    tri_b = jnp.where(tri, 1.0, 0.0).astype(jnp.bfloat16)
    parts = []
    off = jnp.float32(0.0)
    for c in range(SEQ // ch):
        chunk = m_row[:, c * ch:(c + 1) * ch]
        p = jnp.dot(chunk.astype(jnp.bfloat16), tri_b,
                    preferred_element_type=jnp.float32)
        parts.append(p + off)
        off = off + jnp.sum(chunk)
    return jnp.concatenate(parts, axis=1), off


def _r2_body(lg_ref, al_ref, selg_ref, wv_ref, aux_ref):
    iota_row = lax.broadcasted_iota(jnp.int32, (1, SEQ), 1)
    iota_row_f = iota_row.astype(jnp.float32)
    qcol = lax.broadcasted_iota(jnp.int32, (KPAD, 1), 0)
    qcol_f = qcol.astype(jnp.float32)
    masks = []
    for b in range(B):
        v_row = lg_ref[b:b + 1, :]
        bits = lax.bitcast_convert_type(v_row, jnp.int32)
        skey = jnp.where(bits >= 0, bits, bits ^ jnp.int32(0x7FFFFFFF))
        def bis(_, lohi):
            lo, hi = lohi
            mid = (lo >> 1) + (hi >> 1) + (lo & hi & 1)
            cnt = jnp.sum(jnp.where(skey > mid, 1, 0))
            big = cnt >= TOPK
            return jnp.where(big, mid + 1, lo), jnp.where(big, hi, mid)
        lo, hi = lax.fori_loop(0, 32, bis,
                               (jnp.int32(-2147483648), jnp.int32(2147483647)))
        tstar = lo
        gt_row = skey > tstar
        eq_row = skey == tstar
        c_gt = jnp.sum(jnp.where(gt_row, 1, 0))
        need_eq = jnp.float32(TOPK) - c_gt.astype(jnp.float32)
        eq_pref, _ = _row_exclusive_prefix(jnp.where(eq_row, 1.0, 0.0))
        mask_row = gt_row | (eq_row & (eq_pref < need_eq))
        masks.append(mask_row)
        pos_row, _ = _row_exclusive_prefix(jnp.where(mask_row, 1.0, 0.0))
        oh = jnp.where((pos_row == qcol_f) & mask_row, 1.0, 0.0)
        sel_col = jnp.sum(oh * iota_row_f, axis=1, keepdims=True)
        vsel_col = jnp.sum(oh * v_row, axis=1, keepdims=True)
        vsel_row = jnp.transpose(vsel_col)
        q_row_f = jnp.transpose(qcol_f)
        valid_row = q_row_f < TOPK
        cmp = valid_row & ((vsel_row > vsel_col)
                           | ((vsel_row == vsel_col) & (q_row_f < qcol_f)))
        rank_col = jnp.sum(jnp.where(cmp, 1.0, 0.0), axis=1, keepdims=True)
        rank_col = jnp.where(qcol_f < TOPK, rank_col, 9999.0)
        rank_row = jnp.transpose(rank_col)
        oh2 = jnp.where(rank_row == qcol_f, 1.0, 0.0)
        wv_col = jnp.sum(oh2 * vsel_row, axis=1, keepdims=True)
        sel0 = jnp.sum(jnp.where(qcol_f == 0.0, sel_col, 0.0), axis=0,
                       keepdims=True)
        wv0 = jnp.sum(jnp.where(qcol_f == 0.0, wv_col, 0.0), axis=0,
                      keepdims=True)
        in_k = qcol_f < TOPK
        sel_col = jnp.where(in_k, sel_col, sel0)
        wv_col = jnp.where(in_k, wv_col, wv0)
        selg_ref[:, b:b + 1] = (sel_col + jnp.float32(b * SEQ)).astype(jnp.int32)
        wv_ref[:, b:b + 1] = wv_col
    al = al_ref[...]
    p = jnp.clip(1.0 / (1.0 + jnp.exp(-al)), 1e-7, 1.0 - 1e-7)
    log1m = jnp.log(1.0 - p)
    s_all = jnp.sum(jnp.sum(log1m, axis=1, keepdims=True), axis=0,
                    keepdims=True)
    union = masks[0] | masks[1]
    c0 = jnp.log(p[0:1, :]) - log1m[0:1, :]
    corr = jnp.sum(jnp.where(union, c0, 0.0), axis=1, keepdims=True)
    aux_ref[...] = -(s_all + corr) / jnp.float32(NTOK)


def _r2(lg, al):
    return pl.pallas_call(
        _r2_body,
        grid=(1,),
        in_specs=[
            pl.BlockSpec((B, SEQ), lambda i: (0, 0)),
            pl.BlockSpec((B, SEQ), lambda i: (0, 0)),
        ],
        out_specs=[
            pl.BlockSpec((KPAD, B), lambda i: (0, 0)),
            pl.BlockSpec((KPAD, B), lambda i: (0, 0)),
            pl.BlockSpec((1, 1), lambda i: (0, 0)),
        ],
        out_shape=[
            jax.ShapeDtypeStruct((KPAD, B), jnp.int32),
            jax.ShapeDtypeStruct((KPAD, B), jnp.float32),
            jax.ShapeDtypeStruct((1, 1), jnp.float32),
        ],
    )(lg, al)


def _gather(xf, idx_flat):
    mesh = plsc.VectorSubcoreMesh(core_axis_name="c", subcore_axis_name="s")

    @functools.partial(
        pl.kernel,
        mesh=mesh,
        out_type=jax.ShapeDtypeStruct((B * KPAD, D), jnp.float32),
        scratch_types=[
            pltpu.VMEM((_BPW,), jnp.int32),
            pltpu.VMEM((_BPW, D), jnp.float32),
            pltpu.SemaphoreType.DMA,
        ],
    )
    def g(x_hbm, idx_hbm, fx_hbm, idx_v, rows_v, sem):
        wid = lax.axis_index("s") * 2 + lax.axis_index("c")
        base = wid * _BPW
        pltpu.sync_copy(idx_hbm.at[pl.ds(base, _BPW)], idx_v)
        pltpu.async_copy(x_hbm.at[idx_v], rows_v, sem).wait()
        pltpu.sync_copy(rows_v, fx_hbm.at[pl.ds(base, _BPW)])

    return g(xf, idx_flat)


def _m_body(fx_ref, w1_ref, w3_ref, w2_ref, wv_ref, o_ref):
    h = pl.program_id(0)
    fxb = fx_ref[...]
    fb = fxb.astype(jnp.bfloat16)
    a = jnp.dot(fb, w1_ref[...].astype(jnp.bfloat16),
                preferred_element_type=jnp.float32)
    b3 = jnp.dot(fb, w3_ref[...].astype(jnp.bfloat16),
                 preferred_element_type=jnp.float32)
    g = a * (1.0 / (1.0 + jnp.exp(-a)))
    t = (g * b3).astype(jnp.bfloat16)
    yp = jnp.dot(t, w2_ref[...].astype(jnp.bfloat16),
                 preferred_element_type=jnp.float32)
    contrib = wv_ref[...] * yp

    @pl.when(h == 0)
    def _():
        o_ref[...] = fxb + contrib

    @pl.when(h != 0)
    def _():
        o_ref[...] = o_ref[...] + contrib


def _m(fx, wvf, W1, W2, W3):
    nh = HID // M_HBLK
    return pl.pallas_call(
        _m_body,
        grid=(nh,),
        in_specs=[
            pl.BlockSpec((B * KPAD, D), lambda h: (0, 0)),
            pl.BlockSpec((D, M_HBLK), lambda h: (0, h)),
            pl.BlockSpec((D, M_HBLK), lambda h: (0, h)),
            pl.BlockSpec((M_HBLK, D), lambda h: (h, 0)),
            pl.BlockSpec((B * KPAD, 1), lambda h: (0, 0)),
        ],
        out_specs=pl.BlockSpec((B * KPAD, D), lambda h: (0, 0)),
        out_shape=jax.ShapeDtypeStruct((B * KPAD, D), jnp.float32),
    )(fx, W1, W3, W2, wvf)


def _scatter_inplace(payload, idx_flat, out_ref):
    mesh = plsc.VectorSubcoreMesh(core_axis_name="c", subcore_axis_name="s")

    @functools.partial(
        pl.kernel,
        mesh=mesh,
        out_type=(),
        scratch_types=[
            pltpu.VMEM((_BPW,), jnp.int32),
            pltpu.VMEM((_BPW, D), jnp.float32),
            pltpu.SemaphoreType.DMA,
        ],
    )
    def s(p_hbm, idx_hbm, o_hbm, idx_v, rows_v, sem):
        wid = lax.axis_index("s") * 2 + lax.axis_index("c")
        base = wid * _BPW
        pltpu.sync_copy(idx_hbm.at[pl.ds(base, _BPW)], idx_v)
        pltpu.sync_copy(p_hbm.at[pl.ds(base, _BPW)], rows_v)
        pltpu.async_copy(rows_v, o_hbm.at[idx_v], sem).wait()

    return s(payload, idx_flat, out_ref)


def kernel(x, Wr, Wa, W1, W2, W3):
    xf = x.reshape(NTOK, D)
    lg_in = (x @ Wr)[..., 0]
    out0, al = _r1(xf, jnp.transpose(Wa))
    selg, wv, aux = _r2(lg_in, al)
    idx_flat = jnp.transpose(selg).reshape(B * KPAD)
    wvf = jnp.transpose(wv).reshape(B * KPAD, 1)
    fx = _gather(xf, idx_flat)
    payload = _m(fx, wvf, W1, W2, W3)
    oref = jax.new_ref(out0)
    _scatter_inplace(payload, idx_flat, oref)
    out = oref[...].reshape(B, SEQ, D)
    return out, aux[0, 0]
